# baseline (speedup 1.0000x reference)
import functools

import jax
import jax.numpy as jnp
from jax import lax
from jax.experimental import pallas as pl
from jax.experimental.pallas import tpu as pltpu
from jax.experimental.pallas import tpu_sc as plsc

NUM_CORES = 2
NUM_SUBCORES = 16
NW = NUM_CORES * NUM_SUBCORES


def _build(B, H, V, D):
    GD = D // 8
    CB = B // 128
    b_per_w = B // NW
    n_idx = b_per_w * H
    assert b_per_w == 512 and H == 50 and D == 32

    mesh = plsc.VectorSubcoreMesh(core_axis_name="c", subcore_axis_name="s")

    @functools.partial(
        pl.kernel,
        out_type=jax.ShapeDtypeStruct((H * GD * CB * 8 * 128,), jnp.float32),
        mesh=mesh,
        scratch_types=[
            pltpu.VMEM((n_idx,), jnp.int32),
            pltpu.VMEM((n_idx,), jnp.int32),
            pltpu.VMEM((2 * 512, 32), jnp.float32),
            pltpu.VMEM((2 * 16384,), jnp.float32),
            [pltpu.SemaphoreType.DMA] * 2,
            [pltpu.SemaphoreType.DMA] * 2,
        ],
        compiler_params=pltpu.CompilerParams(
            use_tc_tiling_on_sc=False, needs_layout_passes=False,
            disable_bounds_checks=True),
    )
    def gather_kernel(idx_hbm, table_hbm, x_hbm, slab, idxT, rows, xbuf,
                      gsems, ssems):
        wid = lax.axis_index("s") * NUM_CORES + lax.axis_index("c")
        iota = lax.iota(jnp.int32, 16)

        pltpu.sync_copy(idx_hbm.at[pl.ds(wid * n_idx, n_idx)], slab)

        v50 = iota * H

        @plsc.parallel_loop(0, 32, unroll=2)
        def _(j0):
            for h in range(H):
                vals = plsc.load_gather(slab, [j0 * (16 * H) + v50 + h])
                idxT[pl.ds(h * 512 + j0 * 16, 16)] = vals

        def gather(h, p):
            return pltpu.make_async_copy(
                table_hbm.at[idxT.at[pl.ds(h * 512, 512)]],
                rows.at[pl.ds(p * 512, 512)],
                gsems[p],
            )

        def store(h, p, g):
            return pltpu.make_async_copy(
                xbuf.at[pl.ds(p * 16384 + g * 4096, 4096)],
                x_hbm.at[pl.ds((h * GD + g) * (CB * 1024) + wid * 4096, 4096)],
                ssems[p],
            )

        cols = [jnp.full((16,), d, jnp.int32) for d in range(D)]

        def transpose(h, p):
            @plsc.parallel_loop(0, 32, unroll=4)
            def _(t):
                ridx = p * 512 + t * 16 + iota
                base = p * 16384 + (t // 8) * 1024 + (t % 8) * 16
                for g in range(GD):
                    for s in range(8):
                        vals = plsc.load_gather(rows, [ridx, cols[g * 8 + s]])
                        xbuf[pl.ds(base + g * 4096 + s * 128, 16)] = vals

        def stage(h, p, first, last):
            gather(h, p).wait()
            if not first:
                for g in range(GD):
                    store(h - 2, p, g).wait()
            transpose(h, p)
            if not last:
                gather(h + 2, p).start()
            for g in range(GD):
                store(h, p, g).start()

        gather(0, 0).start()
        gather(1, 1).start()
        stage(0, 0, True, False)
        stage(1, 1, True, False)

        @pl.loop(2, H - 2, step=2)
        def _(h0):
            stage(h0, 0, False, False)
            stage(h0 + 1, 1, False, False)

        stage(H - 2, 0, False, True)
        stage(H - 1, 1, False, True)
        for g in range(GD):
            store(H - 2, 0, g).wait()
            store(H - 1, 1, g).wait()

    return gather_kernel


@jax.jit
def kernel(lookup, table):
    B, H = lookup.shape
    V, D = table.shape
    idx_flat = lookup.reshape(B * H).astype(jnp.int32)
    x = _build(B, H, V, D)(idx_flat, table)
    x5 = x.reshape(H, D // 8, B // 128, 8, 128)
    return jnp.transpose(x5, (2, 4, 0, 1, 3)).reshape(B, H, D)


# speedup vs baseline: 1.0696x; 1.0696x over previous
import functools

import jax
import jax.numpy as jnp
from jax import lax
from jax.experimental import pallas as pl
from jax.experimental.pallas import tpu as pltpu
from jax.experimental.pallas import tpu_sc as plsc

NUM_CORES = 2
NUM_SUBCORES = 16
NW = NUM_CORES * NUM_SUBCORES


def _build(B, H, V, D):
    GD = D // 8
    CB = B // 128
    b_per_w = B // NW
    n_idx = b_per_w * H
    assert b_per_w == 512 and H == 50 and D == 32

    mesh = plsc.VectorSubcoreMesh(core_axis_name="c", subcore_axis_name="s")

    @functools.partial(
        pl.kernel,
        out_type=jax.ShapeDtypeStruct((H * GD * CB * 8 * 128,), jnp.float32),
        mesh=mesh,
        scratch_types=[
            pltpu.VMEM((n_idx,), jnp.int32),
            pltpu.VMEM((n_idx,), jnp.int32),
            pltpu.VMEM((2 * 512, 32), jnp.float32),
            pltpu.VMEM((2 * 16384,), jnp.float32),
            [pltpu.SemaphoreType.DMA] * 2,
            [pltpu.SemaphoreType.DMA] * 2,
        ],
        compiler_params=pltpu.CompilerParams(
            use_tc_tiling_on_sc=False, needs_layout_passes=False,
            disable_bounds_checks=True),
    )
    def gather_kernel(idx_hbm, table_hbm, x_hbm, slab, idxT, rows, xbuf,
                      gsems, ssems):
        wid = lax.axis_index("s") * NUM_CORES + lax.axis_index("c")
        iota = lax.iota(jnp.int32, 16)

        pltpu.sync_copy(idx_hbm.at[pl.ds(wid * n_idx, n_idx)], slab)

        v50 = iota * H

        @plsc.parallel_loop(0, 32, unroll=2)
        def _(j0):
            for h in range(H):
                vals = plsc.load_gather(slab, [j0 * (16 * H) + v50 + h])
                idxT[pl.ds(h * 512 + j0 * 16, 16)] = vals

        def gather(h, p):
            return pltpu.make_async_copy(
                table_hbm.at[idxT.at[pl.ds(h * 512, 512)]],
                rows.at[pl.ds(p * 512, 512)],
                gsems[p],
            )

        def store(h, p, g):
            return pltpu.make_async_copy(
                xbuf.at[pl.ds(p * 16384 + g * 4096, 4096)],
                x_hbm.at[pl.ds((h * GD + g) * (CB * 1024) + wid * 4096, 4096)],
                ssems[p],
            )

        cols = [jnp.full((16,), d, jnp.int32) for d in range(D)]

        def transpose(h, p):
            @plsc.parallel_loop(0, 32, unroll=2)
            def _(t):
                ridx = p * 512 + t * 16 + iota
                base = p * 16384 + (t // 8) * 1024 + (t % 8) * 16
                for g in range(GD):
                    for s in range(8):
                        vals = plsc.load_gather(rows, [ridx, cols[g * 8 + s]])
                        xbuf[pl.ds(base + g * 4096 + s * 128, 16)] = vals

        def stage(h, p, first, last):
            gather(h, p).wait()
            if not first:
                for g in range(GD):
                    store(h - 2, p, g).wait()
            transpose(h, p)
            if not last:
                gather(h + 2, p).start()
            for g in range(GD):
                store(h, p, g).start()

        gather(0, 0).start()
        gather(1, 1).start()
        stage(0, 0, True, False)
        stage(1, 1, True, False)

        @pl.loop(2, H - 2, step=2)
        def _(h0):
            stage(h0, 0, False, False)
            stage(h0 + 1, 1, False, False)

        stage(H - 2, 0, False, True)
        stage(H - 1, 1, False, True)
        for g in range(GD):
            store(H - 2, 0, g).wait()
            store(H - 1, 1, g).wait()

    return gather_kernel


@jax.jit
def kernel(lookup, table):
    B, H = lookup.shape
    V, D = table.shape
    idx_flat = lookup.reshape(B * H).astype(jnp.int32)
    x = _build(B, H, V, D)(idx_flat, table)
    x5 = x.reshape(H, D // 8, B // 128, 8, 128)
    return jnp.transpose(x5, (2, 4, 0, 1, 3)).reshape(B, H, D)


# speedup vs baseline: 1.4203x; 1.3279x over previous
import functools

import jax
import jax.numpy as jnp
from jax import lax
from jax.experimental import pallas as pl
from jax.experimental.pallas import tpu as pltpu
from jax.experimental.pallas import tpu_sc as plsc

NUM_CORES = 2
NUM_SUBCORES = 16
NW = NUM_CORES * NUM_SUBCORES


def _build(B, H, V, D):
    GD = D // 8
    CB = B // 128
    b_per_w = B // NW
    n_idx = b_per_w * H
    assert b_per_w == 512 and H == 50 and D == 32

    mesh = plsc.VectorSubcoreMesh(core_axis_name="c", subcore_axis_name="s")

    @functools.partial(
        pl.kernel,
        out_type=jax.ShapeDtypeStruct((H * GD * CB * 8 * 128,), jnp.float32),
        mesh=mesh,
        scratch_types=[
            pltpu.VMEM((n_idx,), jnp.int32),
            pltpu.VMEM((n_idx,), jnp.int32),
            pltpu.VMEM((2 * 512, 32), jnp.float32),
            pltpu.VMEM((2 * 16384,), jnp.float32),
            [pltpu.SemaphoreType.DMA] * 2,
            [pltpu.SemaphoreType.DMA] * 2,
        ],
        compiler_params=pltpu.CompilerParams(
            use_tc_tiling_on_sc=False, needs_layout_passes=False,
            disable_bounds_checks=True),
    )
    def gather_kernel(idx_hbm, table_hbm, x_hbm, slab, idxT, rows, xbuf,
                      gsems, ssems):
        wid = lax.axis_index("s") * NUM_CORES + lax.axis_index("c")
        iota = lax.iota(jnp.int32, 16)

        pltpu.sync_copy(idx_hbm.at[pl.ds(wid * n_idx, n_idx)], slab)

        v50 = iota * H

        @plsc.parallel_loop(0, 32, unroll=2)
        def _(j0):
            for h in range(H):
                vals = plsc.load_gather(slab, [j0 * (16 * H) + v50 + h])
                r32 = ((vals & -8192) | ((vals & 2047) << 2)
                       | ((vals >> 11) & 3))
                idxT[pl.ds(h * 512 + j0 * 16, 16)] = r32

        def gather(h, p):
            return pltpu.make_async_copy(
                table_hbm.at[idxT.at[pl.ds(h * 512, 512)]],
                rows.at[pl.ds(p * 512, 512)],
                gsems[p],
            )

        def store(h, p, g):
            return pltpu.make_async_copy(
                xbuf.at[pl.ds(p * 16384 + g * 4096, 4096)],
                x_hbm.at[pl.ds((h * GD + g) * (CB * 1024) + wid * 4096, 4096)],
                ssems[p],
            )

        cols = [jnp.full((16,), d, jnp.int32) for d in range(D)]

        def transpose(h, p):
            @plsc.parallel_loop(0, 32, unroll=2)
            def _(t):
                ridx = p * 512 + t * 16 + iota
                base = p * 16384 + (t // 8) * 1024 + (t % 8) * 16
                for g in range(GD):
                    for s in range(8):
                        vals = plsc.load_gather(rows, [ridx, cols[g * 8 + s]])
                        xbuf[pl.ds(base + g * 4096 + s * 128, 16)] = vals

        def stage(h, p, first, last):
            gather(h, p).wait()
            if not first:
                for g in range(GD):
                    store(h - 2, p, g).wait()
            transpose(h, p)
            if not last:
                gather(h + 2, p).start()
            for g in range(GD):
                store(h, p, g).start()

        gather(0, 0).start()
        gather(1, 1).start()
        stage(0, 0, True, False)
        stage(1, 1, True, False)

        @pl.loop(2, H - 2, step=2)
        def _(h0):
            stage(h0, 0, False, False)
            stage(h0 + 1, 1, False, False)

        stage(H - 2, 0, False, True)
        stage(H - 1, 1, False, True)
        for g in range(GD):
            store(H - 2, 0, g).wait()
            store(H - 1, 1, g).wait()

    return gather_kernel


_BC = 8192


def _tc_transpose(table):
    V, D = table.shape
    nblk = -(-V // _BC)
    tt = table.T

    def body(tt_ref, out_ref):
        t = tt_ref[...].T
        q = _BC // 4
        out_ref[...] = jnp.concatenate(
            [t[k * q:(k + 1) * q, :] for k in range(4)], axis=1)

    return pl.pallas_call(
        body,
        grid=(nblk,),
        in_specs=[pl.BlockSpec((D, _BC), lambda i: (0, i))],
        out_specs=pl.BlockSpec((_BC // 4, 128), lambda i: (i, 0)),
        out_shape=jax.ShapeDtypeStruct((nblk * _BC // 4, 128), jnp.float32),
    )(tt)


@jax.jit
def kernel(lookup, table):
    B, H = lookup.shape
    V, D = table.shape
    idx_flat = lookup.reshape(B * H).astype(jnp.int32)
    t_rm = _tc_transpose(table)
    Vp = t_rm.shape[0] * t_rm.shape[1] // D
    t_rm = t_rm.reshape(Vp, D)
    x = _build(B, H, Vp, D)(idx_flat, t_rm)
    x5 = x.reshape(H, D // 8, B // 128, 8, 128)
    return jnp.transpose(x5, (2, 4, 0, 1, 3)).reshape(B, H, D)


# speedup vs baseline: 1.4286x; 1.0059x over previous
import functools

import jax
import jax.numpy as jnp
from jax import lax
from jax.experimental import pallas as pl
from jax.experimental.pallas import tpu as pltpu
from jax.experimental.pallas import tpu_sc as plsc

NUM_CORES = 2
NUM_SUBCORES = 16
NW = NUM_CORES * NUM_SUBCORES


def _build(B, H, V, D):
    GD = D // 8
    CB = B // 128
    b_per_w = B // NW
    n_idx = b_per_w * H
    assert b_per_w == 512 and H == 50 and D == 32

    mesh = plsc.VectorSubcoreMesh(core_axis_name="c", subcore_axis_name="s")

    @functools.partial(
        pl.kernel,
        out_type=jax.ShapeDtypeStruct((H * GD * CB * 8 * 128,), jnp.float32),
        mesh=mesh,
        scratch_types=[
            pltpu.VMEM((n_idx,), jnp.int32),
            pltpu.VMEM((n_idx,), jnp.int32),
            pltpu.VMEM((2 * 512, 32), jnp.float32),
            pltpu.VMEM((2 * 16384,), jnp.float32),
            [pltpu.SemaphoreType.DMA] * 2,
            [pltpu.SemaphoreType.DMA] * 2,
        ],
        compiler_params=pltpu.CompilerParams(
            use_tc_tiling_on_sc=False, needs_layout_passes=False,
            disable_bounds_checks=True),
    )
    def gather_kernel(idx_hbm, table_hbm, x_hbm, slab, idxT, rows, xbuf,
                      gsems, ssems):
        wid = lax.axis_index("s") * NUM_CORES + lax.axis_index("c")
        iota = lax.iota(jnp.int32, 16)

        pltpu.sync_copy(idx_hbm.at[pl.ds(wid * n_idx, n_idx)], slab)

        v50 = iota * H

        q = _BC // 4
        qbits = q.bit_length() - 1

        @plsc.parallel_loop(0, 32, unroll=2)
        def _(j0):
            for h in range(H):
                vals = plsc.load_gather(slab, [j0 * (16 * H) + v50 + h])
                r32 = ((vals & -_BC) | ((vals & (q - 1)) << 2)
                       | ((vals >> qbits) & 3))
                idxT[pl.ds(h * 512 + j0 * 16, 16)] = r32

        def gather(h, p):
            return pltpu.make_async_copy(
                table_hbm.at[idxT.at[pl.ds(h * 512, 512)]],
                rows.at[pl.ds(p * 512, 512)],
                gsems[p],
            )

        def store(h, p, g):
            return pltpu.make_async_copy(
                xbuf.at[pl.ds(p * 16384 + g * 4096, 4096)],
                x_hbm.at[pl.ds((h * GD + g) * (CB * 1024) + wid * 4096, 4096)],
                ssems[p],
            )

        cols = [jnp.full((16,), d, jnp.int32) for d in range(D)]

        def transpose(h, p):
            @plsc.parallel_loop(0, 32, unroll=2)
            def _(t):
                ridx = p * 512 + t * 16 + iota
                base = p * 16384 + (t // 8) * 1024 + (t % 8) * 16
                for g in range(GD):
                    for s in range(8):
                        vals = plsc.load_gather(rows, [ridx, cols[g * 8 + s]])
                        xbuf[pl.ds(base + g * 4096 + s * 128, 16)] = vals

        def stage(h, p, first, last):
            gather(h, p).wait()
            if not first:
                for g in range(GD):
                    store(h - 2, p, g).wait()
            transpose(h, p)
            if not last:
                gather(h + 2, p).start()
            for g in range(GD):
                store(h, p, g).start()

        gather(0, 0).start()
        gather(1, 1).start()
        stage(0, 0, True, False)
        stage(1, 1, True, False)

        @pl.loop(2, H - 2, step=2)
        def _(h0):
            stage(h0, 0, False, False)
            stage(h0 + 1, 1, False, False)

        stage(H - 2, 0, False, True)
        stage(H - 1, 1, False, True)
        for g in range(GD):
            store(H - 2, 0, g).wait()
            store(H - 1, 1, g).wait()

    return gather_kernel


_BC = 16384


def _tc_transpose(table):
    V, D = table.shape
    nblk = -(-V // _BC)
    tt = table.T

    def body(tt_ref, out_ref):
        t = tt_ref[...].T
        q = _BC // 4
        out_ref[...] = jnp.concatenate(
            [t[k * q:(k + 1) * q, :] for k in range(4)], axis=1)

    return pl.pallas_call(
        body,
        grid=(nblk,),
        in_specs=[pl.BlockSpec((D, _BC), lambda i: (0, i))],
        out_specs=pl.BlockSpec((_BC // 4, 128), lambda i: (i, 0)),
        out_shape=jax.ShapeDtypeStruct((nblk * _BC // 4, 128), jnp.float32),
    )(tt)


@jax.jit
def kernel(lookup, table):
    B, H = lookup.shape
    V, D = table.shape
    idx_flat = lookup.reshape(B * H).astype(jnp.int32)
    t_rm = _tc_transpose(table)
    Vp = t_rm.shape[0] * t_rm.shape[1] // D
    t_rm = t_rm.reshape(Vp, D)
    x = _build(B, H, Vp, D)(idx_flat, t_rm)
    x5 = x.reshape(H, D // 8, B // 128, 8, 128)
    return jnp.transpose(x5, (2, 4, 0, 1, 3)).reshape(B, H, D)
